# baseline (speedup 1.0000x reference)
import functools

import jax
import jax.numpy as jnp
from jax import lax
from jax.experimental import pallas as pl
from jax.experimental.pallas import tpu as pltpu
from jax.experimental.pallas import tpu_sc as plsc


def _build_gather(B, S, L, Dp, Dc, Dg, nc, nw):
    bw = B // nw
    CH, NBUF = 2, 2
    n_ch = bw // CH
    mesh = plsc.VectorSubcoreMesh(core_axis_name="core", subcore_axis_name="sub")

    @functools.partial(
        pl.kernel, mesh=mesh,
        out_type=[
            jax.ShapeDtypeStruct((B, Dp, L), jnp.float32),
            jax.ShapeDtypeStruct((B, Dc, L), jnp.float32),
            jax.ShapeDtypeStruct((B, Dg, L), jnp.float32),
        ],
        scratch_types=[
            pltpu.VMEM((bw,), jnp.int32),
            pltpu.VMEM((n_ch, CH), jnp.int32),
            pltpu.VMEM((bw, Dp, L), jnp.float32),
            pltpu.VMEM((NBUF, CH, Dc, L), jnp.float32),
            pltpu.VMEM((bw, Dg, L), jnp.float32),
            pltpu.SemaphoreType.DMA,
            pltpu.SemaphoreType.DMA,
            pltpu.SemaphoreType.DMA,
            pltpu.SemaphoreType.DMA,
        ],
    )
    def run(idx_hbm, idx2_hbm, p_hbm, c_hbm, g_hbm, po_hbm, co_hbm, go_hbm,
            idx_v, idx2_v, p_v, c_v, g_v, gsem, wsem, psem, qsem):
        wid = lax.axis_index("sub") * nc + lax.axis_index("core")
        base = wid * bw
        pltpu.sync_copy(idx_hbm.at[pl.ds(base, bw)], idx_v)
        pltpu.sync_copy(
            idx2_hbm.at[pl.ds(pl.multiple_of(base // CH, 8), n_ch)], idx2_v)

        p_in = pltpu.async_copy(p_hbm.at[idx_v], p_v, psem)
        g_in = pltpu.async_copy(g_hbm.at[idx_v], g_v, qsem)

        def gather_chunk(j, b):
            return pltpu.async_copy(c_hbm.at[idx2_v.at[j]], c_v.at[b], gsem)

        def write_chunk(j, b):
            return pltpu.async_copy(
                c_v.at[b], co_hbm.at[pl.ds(base + j * CH, CH)], wsem)

        def drain_gather(b):
            pltpu.make_async_copy(c_hbm.at[idx2_v.at[0]], c_v.at[b], gsem).wait()

        def drain_write(b):
            pltpu.make_async_copy(
                c_v.at[b], co_hbm.at[pl.ds(base, CH)], wsem).wait()

        for b in range(NBUF):
            gather_chunk(b, b)

        def body(j, carry):
            b = lax.rem(j, NBUF)
            drain_gather(b)
            write_chunk(j, b)
            drain_write(b)
            gather_chunk(j + NBUF, b)
            return carry

        lax.fori_loop(0, n_ch - NBUF, body, 0)

        for j in range(n_ch - NBUF, n_ch):
            drain_gather(j % NBUF)
            write_chunk(j, j % NBUF)

        p_in.wait()
        pltpu.sync_copy(p_v, po_hbm.at[pl.ds(base, bw)])
        g_in.wait()
        pltpu.sync_copy(g_v, go_hbm.at[pl.ds(base, bw)])

        for j in range(NBUF):
            drain_write(0)

    return run, CH


def kernel(idx, p, c, g):
    S, L, Dc = c.shape
    Dp = p.shape[2]
    Dg = g.shape[2]
    B = idx.shape[0]

    info = plsc.get_sparse_core_info()
    nc = info.num_cores
    nw = nc * info.num_subcores

    run, CH = _build_gather(B, S, L, Dp, Dc, Dg, nc, nw)

    pt = jnp.transpose(p, (0, 2, 1))
    ct = jnp.transpose(c, (0, 2, 1))
    gt = jnp.transpose(g, (0, 2, 1))
    idx2 = idx.reshape(B // CH, CH)
    pot, cot, got = run(idx, idx2, pt, ct, gt)
    return (jnp.transpose(pot, (0, 2, 1)),
            jnp.transpose(cot, (0, 2, 1)),
            jnp.transpose(got, (0, 2, 1)))


# speedup vs baseline: 1.0031x; 1.0031x over previous
import functools

import jax
import jax.numpy as jnp
from jax import lax
from jax.experimental import pallas as pl
from jax.experimental.pallas import tpu as pltpu
from jax.experimental.pallas import tpu_sc as plsc


def _build_gather(B, S, L, Dp, Dc, Dg, nc, nw):
    bw = B // nw
    CH, NBUF = 2, 3
    n_ch = bw // CH
    mesh = plsc.VectorSubcoreMesh(core_axis_name="core", subcore_axis_name="sub")

    @functools.partial(
        pl.kernel, mesh=mesh,
        out_type=[
            jax.ShapeDtypeStruct((B, Dp, L), jnp.float32),
            jax.ShapeDtypeStruct((B, Dc, L), jnp.float32),
            jax.ShapeDtypeStruct((B, Dg, L), jnp.float32),
        ],
        scratch_types=[
            pltpu.VMEM((bw,), jnp.int32),
            pltpu.VMEM((n_ch, CH), jnp.int32),
            pltpu.VMEM((bw, Dp, L), jnp.float32),
            pltpu.VMEM((NBUF, CH, Dc, L), jnp.float32),
            pltpu.VMEM((bw, Dg, L), jnp.float32),
            pltpu.SemaphoreType.DMA,
            pltpu.SemaphoreType.DMA,
            pltpu.SemaphoreType.DMA,
            pltpu.SemaphoreType.DMA,
        ],
    )
    def run(idx_hbm, idx2_hbm, p_hbm, c_hbm, g_hbm, po_hbm, co_hbm, go_hbm,
            idx_v, idx2_v, p_v, c_v, g_v, gsem, wsem, psem, qsem):
        wid = lax.axis_index("sub") * nc + lax.axis_index("core")
        base = wid * bw
        pltpu.sync_copy(idx_hbm.at[pl.ds(base, bw)], idx_v)
        pltpu.sync_copy(
            idx2_hbm.at[pl.ds(pl.multiple_of(base // CH, 8), n_ch)], idx2_v)

        gathers = [None] * n_ch
        writes = [None] * n_ch
        p_in = g_in = None
        for i in range(n_ch):
            b = i % NBUF
            if i >= NBUF:
                writes[i - NBUF].wait()
            gathers[i] = pltpu.async_copy(
                c_hbm.at[idx2_v.at[i]], c_v.at[b], gsem)
            if i == NBUF - 1:
                p_in = pltpu.async_copy(p_hbm.at[idx_v], p_v, psem)
                g_in = pltpu.async_copy(g_hbm.at[idx_v], g_v, qsem)
            if i >= 1:
                gathers[i - 1].wait()
                writes[i - 1] = pltpu.async_copy(
                    c_v.at[(i - 1) % NBUF],
                    co_hbm.at[pl.ds(base + (i - 1) * CH, CH)], wsem)
        gathers[n_ch - 1].wait()
        writes[n_ch - 1] = pltpu.async_copy(
            c_v.at[(n_ch - 1) % NBUF],
            co_hbm.at[pl.ds(base + (n_ch - 1) * CH, CH)], wsem)

        p_in.wait()
        pltpu.sync_copy(p_v, po_hbm.at[pl.ds(base, bw)])
        g_in.wait()
        pltpu.sync_copy(g_v, go_hbm.at[pl.ds(base, bw)])

        writes[n_ch - 2].wait()
        writes[n_ch - 1].wait()

    return run, CH


def kernel(idx, p, c, g):
    S, L, Dc = c.shape
    Dp = p.shape[2]
    Dg = g.shape[2]
    B = idx.shape[0]

    info = plsc.get_sparse_core_info()
    nc = info.num_cores
    nw = nc * info.num_subcores

    run, CH = _build_gather(B, S, L, Dp, Dc, Dg, nc, nw)

    pt = jnp.transpose(p, (0, 2, 1))
    ct = jnp.transpose(c, (0, 2, 1))
    gt = jnp.transpose(g, (0, 2, 1))
    idx2 = idx.reshape(B // CH, CH)
    pot, cot, got = run(idx, idx2, pt, ct, gt)
    return (jnp.transpose(pot, (0, 2, 1)),
            jnp.transpose(cot, (0, 2, 1)),
            jnp.transpose(got, (0, 2, 1)))


# speedup vs baseline: 1.0094x; 1.0063x over previous
import functools

import jax
import jax.numpy as jnp
from jax import lax
from jax.experimental import pallas as pl
from jax.experimental.pallas import tpu as pltpu
from jax.experimental.pallas import tpu_sc as plsc


def _build_gather(B, S, L, Dp, Dc, Dg, nc, nw):
    bw = B // nw
    CH, NBUF = 2, 3
    n_ch = bw // CH
    mesh = plsc.VectorSubcoreMesh(core_axis_name="core", subcore_axis_name="sub")

    @functools.partial(
        pl.kernel, mesh=mesh,
        out_type=[
            jax.ShapeDtypeStruct((B, Dp, L), jnp.float32),
            jax.ShapeDtypeStruct((B, Dc, L), jnp.float32),
            jax.ShapeDtypeStruct((B, Dg, L), jnp.float32),
        ],
        scratch_types=[
            pltpu.VMEM((bw,), jnp.int32),
            pltpu.VMEM((n_ch, CH), jnp.int32),
            pltpu.VMEM((bw, Dp, L), jnp.float32),
            pltpu.VMEM((NBUF, CH, Dc, L), jnp.float32),
            pltpu.VMEM((bw, Dg, L), jnp.float32),
            pltpu.SemaphoreType.DMA,
            pltpu.SemaphoreType.DMA,
            pltpu.SemaphoreType.DMA,
            pltpu.SemaphoreType.DMA,
        ],
    )
    def run(idx_hbm, idx2_hbm, p_hbm, c_hbm, g_hbm, po_hbm, co_hbm, go_hbm,
            idx_v, idx2_v, p_v, c_v, g_v, gsem, wsem, psem, qsem):
        wid = lax.axis_index("sub") * nc + lax.axis_index("core")
        base = wid * bw
        i1 = pltpu.async_copy(idx_hbm.at[pl.ds(base, bw)], idx_v, psem)
        i2 = pltpu.async_copy(
            idx2_hbm.at[pl.ds(pl.multiple_of(base // CH, 8), n_ch)], idx2_v,
            qsem)
        i1.wait()
        i2.wait()

        gathers = [None] * n_ch
        writes = [None] * n_ch
        p_in = g_in = None
        for i in range(n_ch):
            b = i % NBUF
            if i >= NBUF:
                writes[i - NBUF].wait()
            gathers[i] = pltpu.async_copy(
                c_hbm.at[idx2_v.at[i]], c_v.at[b], gsem)
            if i == NBUF - 1:
                p_in = pltpu.async_copy(p_hbm.at[idx_v], p_v, psem)
                g_in = pltpu.async_copy(g_hbm.at[idx_v], g_v, qsem)
            if i >= 1:
                gathers[i - 1].wait()
                writes[i - 1] = pltpu.async_copy(
                    c_v.at[(i - 1) % NBUF],
                    co_hbm.at[pl.ds(base + (i - 1) * CH, CH)], wsem)
        gathers[n_ch - 1].wait()
        writes[n_ch - 1] = pltpu.async_copy(
            c_v.at[(n_ch - 1) % NBUF],
            co_hbm.at[pl.ds(base + (n_ch - 1) * CH, CH)], wsem)

        p_in.wait()
        pltpu.sync_copy(p_v, po_hbm.at[pl.ds(base, bw)])
        g_in.wait()
        pltpu.sync_copy(g_v, go_hbm.at[pl.ds(base, bw)])

        writes[n_ch - 2].wait()
        writes[n_ch - 1].wait()

    return run, CH


def kernel(idx, p, c, g):
    S, L, Dc = c.shape
    Dp = p.shape[2]
    Dg = g.shape[2]
    B = idx.shape[0]

    info = plsc.get_sparse_core_info()
    nc = info.num_cores
    nw = nc * info.num_subcores

    run, CH = _build_gather(B, S, L, Dp, Dc, Dg, nc, nw)

    pt = jnp.transpose(p, (0, 2, 1))
    ct = jnp.transpose(c, (0, 2, 1))
    gt = jnp.transpose(g, (0, 2, 1))
    idx2 = idx.reshape(B // CH, CH)
    pot, cot, got = run(idx, idx2, pt, ct, gt)
    return (jnp.transpose(pot, (0, 2, 1)),
            jnp.transpose(cot, (0, 2, 1)),
            jnp.transpose(got, (0, 2, 1)))
